# Initial kernel scaffold; baseline (speedup 1.0000x reference)
import jax
import jax.numpy as jnp
from jax.experimental import pallas as pl


def kernel(hidden_states, adj_i, adj_ii, adj_iii, adj_a, W_i, W_ii, W_iii, W_a):
    raise NotImplementedError("write your pallas kernel here")


# speedup vs baseline: 2.8341x; 2.8341x over previous
import functools

import jax
import jax.numpy as jnp
from jax import lax
from jax.experimental import pallas as pl
from jax.experimental.pallas import tpu as pltpu
from jax.experimental.pallas import tpu_sc as plsc

N = 100000
D = 128
E = 400000
M = 4 * E

NC = 2
NS = 16
LANES = 16

C = 12800
P = 8
NPAD = P * C
PASSES_PER_CORE = P // NC

EPT = M // NS
BLK = 2000
NBLK = EPT // BLK
VPB = BLK // LANES
CAP = BLK + 160
GCH = 128
RPT = C // NS
ZROWS = 16


def _matmul_body(x_ref, w_ref, o_ref):
    o_ref[...] = jnp.dot(x_ref[...], w_ref[...],
                         preferred_element_type=jnp.float32)


def _matmul(x, w_cat):
    BM = 2000
    return pl.pallas_call(
        _matmul_body,
        grid=(N // BM,),
        in_specs=[
            pl.BlockSpec((BM, D), lambda i: (i, 0)),
            pl.BlockSpec((D, 4 * D), lambda i: (0, 0)),
        ],
        out_specs=pl.BlockSpec((BM, 4 * D), lambda i: (i, 0)),
        out_shape=jax.ShapeDtypeStruct((N, 4 * D), jnp.float32),
    )(x, w_cat)


_mesh = plsc.VectorSubcoreMesh(core_axis_name="c", subcore_axis_name="s")


@functools.partial(
    pl.kernel,
    out_type=jax.ShapeDtypeStruct((NPAD, D), jnp.float32),
    mesh=_mesh,
    scratch_types=[
        pltpu.VMEM((BLK,), jnp.int32),
        pltpu.VMEM((BLK,), jnp.int32),
        pltpu.VMEM((CAP,), jnp.int32),
        pltpu.VMEM((CAP,), jnp.int32),
        pltpu.VMEM((GCH,), jnp.int32),
        pltpu.VMEM((GCH, D), jnp.float32),
        pltpu.VMEM((ZROWS, D), jnp.float32),
        pltpu.VMEM_SHARED((C + 8, D), jnp.float32),
        pltpu.SemaphoreType.DMA,
    ],
)
def _edge_aggregate(h_hbm, src_hbm, dst_hbm, out_hbm,
                    srcb, dstb, csrc, cdst, fire_d, rows, zrows,
                    acc, sem):
    core = lax.axis_index("c")
    sub = lax.axis_index("s")
    rel = sub // 4

    zv = jnp.zeros((LANES,), jnp.float32)

    def zfill(i, _):
        zrows[i // 8, pl.ds((i % 8) * LANES, LANES)] = zv
        return 0

    lax.fori_loop(0, ZROWS * 8, zfill, 0)

    def fire_one(f, _):
        off = f * GCH
        for j in range(GCH // LANES):
            fire_d[pl.ds(j * LANES, LANES)] = cdst[pl.ds(off + j * LANES, LANES)]
        pltpu.async_copy(h_hbm.at[csrc.at[pl.ds(off, GCH)]], rows, sem).wait()
        pltpu.sync_copy(rows, acc.at[fire_d], add=True)
        return 0

    for half in range(PASSES_PER_CORE):
        p = core * PASSES_PER_CORE + half
        lo = p * C

        for k in range(RPT // ZROWS):
            pltpu.sync_copy(zrows, acc.at[pl.ds(sub * RPT + k * ZROWS, ZROWS)])
        plsc.subcore_barrier()

        def scan_block(b, cnt):
            base = sub * EPT + b * BLK
            pltpu.sync_copy(src_hbm.at[pl.ds(base, BLK)], srcb)
            pltpu.sync_copy(dst_hbm.at[pl.ds(base, BLK)], dstb)

            def filt(i, cnt):
                d = dstb[pl.ds(i * LANES, LANES)]
                s = srcb[pl.ds(i * LANES, LANES)]
                m = (d >= lo) & (d < lo + C)
                iot = lax.iota(jnp.int32, LANES)
                sc = jnp.where(m, 1, 0)
                for w in (1, 2, 4, 8):
                    sh = jnp.take(sc, jnp.maximum(iot - w, 0))
                    sc = sc + jnp.where(iot >= w, sh, 0)
                total = sc[15]
                j = jnp.zeros((LANES,), jnp.int32)
                tgt = iot + 1
                for w in (8, 4, 2, 1):
                    t2 = j + w
                    sval = jnp.take(sc, t2 - 1)
                    j = jnp.where(sval < tgt, t2, j)
                j = jnp.minimum(j, LANES - 1)
                cdst[pl.ds(cnt, LANES)] = jnp.take(d - lo, j)
                csrc[pl.ds(cnt, LANES)] = jnp.take(s * 4 + rel, j)
                return cnt + total

            cnt = lax.fori_loop(0, VPB, filt, cnt)

            nf = cnt // GCH
            lax.fori_loop(0, nf, fire_one, 0)

            rem_off = nf * GCH
            for j in range(GCH // LANES):
                sv = csrc[pl.ds(rem_off + j * LANES, LANES)]
                dv = cdst[pl.ds(rem_off + j * LANES, LANES)]
                csrc[pl.ds(j * LANES, LANES)] = sv
                cdst[pl.ds(j * LANES, LANES)] = dv
            return cnt - rem_off

        cnt = lax.fori_loop(0, NBLK, scan_block, jnp.int32(0))

        sent_d = jnp.full((LANES,), C, jnp.int32)
        sent_s = jnp.zeros((LANES,), jnp.int32)
        for j in range(GCH // LANES):
            cdst[pl.ds(cnt + j * LANES, LANES)] = sent_d
            csrc[pl.ds(cnt + j * LANES, LANES)] = sent_s
        lax.fori_loop(0, (cnt + GCH) // GCH, fire_one, 0)
        plsc.subcore_barrier()

        pltpu.sync_copy(acc.at[pl.ds(sub * RPT, RPT)],
                        out_hbm.at[pl.ds(lo + sub * RPT, RPT)])
        plsc.subcore_barrier()


def kernel(hidden_states, adj_i, adj_ii, adj_iii, adj_a,
           W_i, W_ii, W_iii, W_a):
    w_cat = jnp.concatenate([W_i, W_ii, W_iii, W_a], axis=1)
    h = _matmul(hidden_states, w_cat)
    h4 = h.reshape(N * 4, D)
    srcs = jnp.concatenate([adj_i[0], adj_ii[0], adj_iii[0], adj_a[0]])
    dsts = jnp.concatenate([adj_i[1], adj_ii[1], adj_iii[1], adj_a[1]])
    out = _edge_aggregate(h4, srcs, dsts)
    return out[:N]


# speedup vs baseline: 3.2374x; 1.1423x over previous
import functools

import jax
import jax.numpy as jnp
from jax import lax
from jax.experimental import pallas as pl
from jax.experimental.pallas import tpu as pltpu
from jax.experimental.pallas import tpu_sc as plsc

N = 100000
D = 128
E = 400000
M = 4 * E

NC = 2
NS = 16
LANES = 16

C = 12544
P = 8
NPAD = P * C
PASSES_PER_CORE = P // NC

EPT = M // NS
BLK = 2000
NBLK = EPT // BLK
VPB = BLK // LANES
GCH = 64
PAIR = 2 * GCH
CAP = BLK + 208
RPT = C // NS


def _matmul_body(x_ref, w_ref, o_ref):
    o_ref[...] = jnp.dot(x_ref[...], w_ref[...],
                         preferred_element_type=jnp.float32)


def _matmul(x, w_cat):
    BM = 2000
    return pl.pallas_call(
        _matmul_body,
        grid=(N // BM,),
        in_specs=[
            pl.BlockSpec((BM, D), lambda i: (i, 0)),
            pl.BlockSpec((D, 4 * D), lambda i: (0, 0)),
        ],
        out_specs=pl.BlockSpec((BM, 4 * D), lambda i: (i, 0)),
        out_shape=jax.ShapeDtypeStruct((N, 4 * D), jnp.float32),
    )(x, w_cat)


_mesh = plsc.VectorSubcoreMesh(core_axis_name="c", subcore_axis_name="s")


@functools.partial(
    pl.kernel,
    out_type=jax.ShapeDtypeStruct((NPAD, D), jnp.float32),
    mesh=_mesh,
    scratch_types=[
        pltpu.VMEM((BLK,), jnp.int32),
        pltpu.VMEM((BLK,), jnp.int32),
        pltpu.VMEM((BLK,), jnp.int32),
        pltpu.VMEM((BLK,), jnp.int32),
        pltpu.VMEM((CAP,), jnp.int32),
        pltpu.VMEM((CAP,), jnp.int32),
        pltpu.VMEM((GCH,), jnp.int32),
        pltpu.VMEM((GCH,), jnp.int32),
        pltpu.VMEM((GCH, D), jnp.float32),
        pltpu.VMEM((GCH, D), jnp.float32),
        pltpu.VMEM_SHARED((C + 8, D), jnp.float32),
        pltpu.SemaphoreType.DMA,
        pltpu.SemaphoreType.DMA,
        pltpu.SemaphoreType.DMA,
        pltpu.SemaphoreType.DMA,
    ],
)
def _edge_aggregate(h_hbm, src_hbm, dst_hbm, zeros_hbm, out_hbm,
                    srcb0, dstb0, srcb1, dstb1, csrc, cdst,
                    fire_d0, fire_d1, rows0, rows1, acc,
                    sem_e0, sem_e1, sem_g0, sem_g1):
    core = lax.axis_index("c")
    sub = lax.axis_index("s")
    rel = sub // 4
    ebase = sub * EPT

    _ring = ((rows0, fire_d0, sem_g0), (rows1, fire_d1, sem_g1))

    def fire_pairs(npair):
        @pl.when(npair >= 1)
        def _():
            pltpu.async_copy(h_hbm.at[csrc.at[pl.ds(0, GCH)]], rows0, sem_g0)
            pltpu.async_copy(h_hbm.at[csrc.at[pl.ds(GCH, GCH)]], rows1,
                             sem_g1)

        def pair_body(q, _):
            for half, (rbuf, fbuf, sem) in enumerate(_ring):
                f = 2 * q + half
                off = f * GCH
                pltpu.make_async_copy(
                    h_hbm.at[csrc.at[pl.ds(off, GCH)]], rbuf, sem).wait()
                for j in range(GCH // LANES):
                    fbuf[pl.ds(j * LANES, LANES)] = (
                        cdst[pl.ds(off + j * LANES, LANES)])
                pltpu.sync_copy(rbuf, acc.at[fbuf], add=True)

                @pl.when(f + 2 < 2 * npair)
                def _():
                    pltpu.async_copy(
                        h_hbm.at[csrc.at[pl.ds(off + PAIR, GCH)]], rbuf, sem)
            return 0

        lax.fori_loop(0, npair, pair_body, 0)

    def carry_remainder(cnt, npair):
        rem_off = npair * PAIR
        for j in range(PAIR // LANES):
            sv = csrc[pl.ds(rem_off + j * LANES, LANES)]
            dv = cdst[pl.ds(rem_off + j * LANES, LANES)]
            csrc[pl.ds(j * LANES, LANES)] = sv
            cdst[pl.ds(j * LANES, LANES)] = dv
        return cnt - rem_off

    for half_pass in range(PASSES_PER_CORE):
        p = core * PASSES_PER_CORE + half_pass
        lo = p * C

        pltpu.sync_copy(zeros_hbm.at[pl.ds(sub * RPT, RPT)],
                        acc.at[pl.ds(sub * RPT, RPT)])
        plsc.subcore_barrier()

        def filter_block(sb, db, cnt):
            def filt(i, cnt):
                d = db[pl.ds(i * LANES, LANES)]
                s = sb[pl.ds(i * LANES, LANES)]
                m = (d >= lo) & (d < lo + C)
                iot = lax.iota(jnp.int32, LANES)
                sc = jnp.where(m, 1, 0)
                for w in (1, 2, 4, 8):
                    sh = jnp.take(sc, jnp.maximum(iot - w, 0))
                    sc = sc + jnp.where(iot >= w, sh, 0)
                total = sc[15]
                j = jnp.zeros((LANES,), jnp.int32)
                tgt = iot + 1
                for w in (8, 4, 2, 1):
                    t2 = j + w
                    sval = jnp.take(sc, t2 - 1)
                    j = jnp.where(sval < tgt, t2, j)
                j = jnp.minimum(j, LANES - 1)
                cdst[pl.ds(cnt, LANES)] = jnp.take(d - lo, j)
                csrc[pl.ds(cnt, LANES)] = jnp.take(s * 4 + rel, j)
                return cnt + total

            return lax.fori_loop(0, VPB, filt, cnt)

        pltpu.async_copy(src_hbm.at[pl.ds(ebase, BLK)], srcb0, sem_e0)
        pltpu.async_copy(dst_hbm.at[pl.ds(ebase, BLK)], dstb0, sem_e0)

        def pair_block(pb, cnt):
            base0 = ebase + 2 * pb * BLK
            base1 = base0 + BLK
            pltpu.make_async_copy(src_hbm.at[pl.ds(base0, BLK)], srcb0,
                                  sem_e0).wait()
            pltpu.make_async_copy(dst_hbm.at[pl.ds(base0, BLK)], dstb0,
                                  sem_e0).wait()
            pltpu.async_copy(src_hbm.at[pl.ds(base1, BLK)], srcb1, sem_e1)
            pltpu.async_copy(dst_hbm.at[pl.ds(base1, BLK)], dstb1, sem_e1)
            cnt = filter_block(srcb0, dstb0, cnt)
            npair = cnt // PAIR
            fire_pairs(npair)
            cnt = carry_remainder(cnt, npair)
            pltpu.make_async_copy(src_hbm.at[pl.ds(base1, BLK)], srcb1,
                                  sem_e1).wait()
            pltpu.make_async_copy(dst_hbm.at[pl.ds(base1, BLK)], dstb1,
                                  sem_e1).wait()

            @pl.when(pb < NBLK // 2 - 1)
            def _():
                base2 = base1 + BLK
                pltpu.async_copy(src_hbm.at[pl.ds(base2, BLK)], srcb0,
                                 sem_e0)
                pltpu.async_copy(dst_hbm.at[pl.ds(base2, BLK)], dstb0,
                                 sem_e0)

            cnt = filter_block(srcb1, dstb1, cnt)
            npair = cnt // PAIR
            fire_pairs(npair)
            cnt = carry_remainder(cnt, npair)
            return cnt

        cnt = lax.fori_loop(0, NBLK // 2, pair_block, jnp.int32(0))

        sent_d = jnp.full((LANES,), C, jnp.int32)
        sent_s = jnp.zeros((LANES,), jnp.int32)
        for j in range(PAIR // LANES):
            cdst[pl.ds(cnt + j * LANES, LANES)] = sent_d
            csrc[pl.ds(cnt + j * LANES, LANES)] = sent_s
        fire_pairs((cnt + PAIR) // PAIR)
        plsc.subcore_barrier()

        pltpu.sync_copy(acc.at[pl.ds(sub * RPT, RPT)],
                        out_hbm.at[pl.ds(lo + sub * RPT, RPT)])
        plsc.subcore_barrier()


def kernel(hidden_states, adj_i, adj_ii, adj_iii, adj_a,
           W_i, W_ii, W_iii, W_a):
    w_cat = jnp.concatenate([W_i, W_ii, W_iii, W_a], axis=1)
    h = _matmul(hidden_states, w_cat)
    h4 = h.reshape(N * 4, D)
    srcs = jnp.concatenate([adj_i[0], adj_ii[0], adj_iii[0], adj_a[0]])
    dsts = jnp.concatenate([adj_i[1], adj_ii[1], adj_iii[1], adj_a[1]])
    zeros = jnp.zeros((C, D), jnp.float32)
    out = _edge_aggregate(h4, srcs, dsts, zeros)
    return out[:N]


# speedup vs baseline: 3.6895x; 1.1396x over previous
import functools

import jax
import jax.numpy as jnp
from jax import lax
from jax.experimental import pallas as pl
from jax.experimental.pallas import tpu as pltpu
from jax.experimental.pallas import tpu_sc as plsc

N = 100000
D = 128
E = 400000
M = 4 * E

NC = 2
NS = 16
LANES = 16

C = 12544
P = 8
NPAD = P * C
PASSES_PER_CORE = P // NC

EPT = M // NS
BLK = 800
NBLK = EPT // BLK
VPB = BLK // 32
GCH = 64
PAIR = 2 * GCH
CAP = BLK + 208
RPT = C // NS


def _matmul_body(x_ref, w_ref, o_ref):
    o_ref[...] = jnp.dot(x_ref[...], w_ref[...],
                         preferred_element_type=jnp.float32)


def _matmul(x, w_cat):
    BM = 2000
    return pl.pallas_call(
        _matmul_body,
        grid=(N // BM,),
        in_specs=[
            pl.BlockSpec((BM, D), lambda i: (i, 0)),
            pl.BlockSpec((D, 4 * D), lambda i: (0, 0)),
        ],
        out_specs=pl.BlockSpec((BM, 4 * D), lambda i: (i, 0)),
        out_shape=jax.ShapeDtypeStruct((N, 4 * D), jnp.float32),
    )(x, w_cat)


_mesh = plsc.VectorSubcoreMesh(core_axis_name="c", subcore_axis_name="s")


@functools.partial(
    pl.kernel,
    out_type=jax.ShapeDtypeStruct((NPAD, D), jnp.float32),
    mesh=_mesh,
    scratch_types=[
        pltpu.VMEM((BLK,), jnp.int32),
        pltpu.VMEM((BLK,), jnp.int32),
        pltpu.VMEM((BLK,), jnp.int32),
        pltpu.VMEM((BLK,), jnp.int32),
        pltpu.VMEM((CAP,), jnp.int32),
        pltpu.VMEM((CAP,), jnp.int32),
        pltpu.VMEM((GCH,), jnp.int32),
        pltpu.VMEM((GCH,), jnp.int32),
        pltpu.VMEM((GCH, D), jnp.float32),
        pltpu.VMEM((GCH, D), jnp.float32),
        pltpu.VMEM_SHARED((C + 8, D), jnp.float32),
        pltpu.SemaphoreType.DMA,
        pltpu.SemaphoreType.DMA,
        pltpu.SemaphoreType.DMA,
        pltpu.SemaphoreType.DMA,
        pltpu.SemaphoreType.DMA,
        pltpu.SemaphoreType.DMA,
    ],
)
def _edge_aggregate(h_hbm, src_hbm, dst_hbm, zeros_hbm, out_hbm,
                    srcb0, dstb0, srcb1, dstb1, csrc, cdst,
                    fire_d0, fire_d1, rows0, rows1, acc,
                    sem_e0, sem_e1, sem_g0, sem_g1, sem_s0, sem_s1):
    core = lax.axis_index("c")
    sub = lax.axis_index("s")
    rel = sub // 4
    ebase = sub * EPT

    def gather_issue(off, rbuf, sem):
        pltpu.async_copy(h_hbm.at[csrc.at[pl.ds(off, GCH)]], rbuf, sem)

    def gather_wait(off, rbuf, sem):
        pltpu.make_async_copy(
            h_hbm.at[csrc.at[pl.ds(off, GCH)]], rbuf, sem).wait()

    def scatter_wait(rbuf, fbuf, sem):
        pltpu.make_async_copy(rbuf, acc.at[fbuf], sem).wait()

    def fire_pairs(npair):
        nf = 2 * npair

        @pl.when(npair >= 1)
        def _():
            gather_issue(0, rows0, sem_g0)

        def pair_body(q, _):
            @pl.when(q >= 1)
            def _():
                scatter_wait(rows1, fire_d1, sem_s1)

            gather_issue(2 * q * GCH + GCH, rows1, sem_g1)
            gather_wait(2 * q * GCH, rows0, sem_g0)
            for j in range(GCH // LANES):
                fire_d0[pl.ds(j * LANES, LANES)] = (
                    cdst[pl.ds(2 * q * GCH + j * LANES, LANES)])
            pltpu.async_copy(rows0, acc.at[fire_d0], sem_s0, add=True)

            @pl.when(q + 1 < npair)
            def _():
                scatter_wait(rows0, fire_d0, sem_s0)
                gather_issue(2 * q * GCH + 2 * GCH, rows0, sem_g0)

            gather_wait(2 * q * GCH + GCH, rows1, sem_g1)
            for j in range(GCH // LANES):
                fire_d1[pl.ds(j * LANES, LANES)] = (
                    cdst[pl.ds(2 * q * GCH + GCH + j * LANES, LANES)])
            pltpu.async_copy(rows1, acc.at[fire_d1], sem_s1, add=True)
            return 0

        lax.fori_loop(0, npair, pair_body, 0)

        @pl.when(npair >= 1)
        def _():
            scatter_wait(rows0, fire_d0, sem_s0)
            scatter_wait(rows1, fire_d1, sem_s1)

    def carry_remainder(cnt, npair):
        rem_off = npair * PAIR
        for j in range(PAIR // LANES):
            sv = csrc[pl.ds(rem_off + j * LANES, LANES)]
            dv = cdst[pl.ds(rem_off + j * LANES, LANES)]
            csrc[pl.ds(j * LANES, LANES)] = sv
            cdst[pl.ds(j * LANES, LANES)] = dv
        return cnt - rem_off

    for half_pass in range(PASSES_PER_CORE):
        p = core * PASSES_PER_CORE + half_pass
        lo = p * C

        pltpu.sync_copy(zeros_hbm.at[pl.ds(sub * RPT, RPT)],
                        acc.at[pl.ds(sub * RPT, RPT)])
        plsc.subcore_barrier()

        def filter_block(sb, db, cnt):
            iot = lax.iota(jnp.int32, LANES)

            def compact16(d, s, cnt):
                m = (d >= lo) & (d < lo + C)
                sc = jnp.where(m, 1, 0)
                for w in (1, 2, 4, 8):
                    sh = jnp.take(sc, jnp.maximum(iot - w, 0))
                    sc = sc + jnp.where(iot >= w, sh, 0)
                total = sc[15]
                j = jnp.zeros((LANES,), jnp.int32)
                tgt = iot + 1
                for w in (8, 4, 2, 1):
                    t2 = j + w
                    sval = jnp.take(sc, t2 - 1)
                    j = jnp.where(sval < tgt, t2, j)
                j = jnp.minimum(j, LANES - 1)
                cdst[pl.ds(cnt, LANES)] = jnp.take(d - lo, j)
                csrc[pl.ds(cnt, LANES)] = jnp.take(s * 4 + rel, j)
                return cnt + total

            def filt(i, cnt):
                da = db[pl.ds(i * 32, LANES)]
                sa = sb[pl.ds(i * 32, LANES)]
                dbv = db[pl.ds(i * 32 + LANES, LANES)]
                sbv = sb[pl.ds(i * 32 + LANES, LANES)]
                cnt = compact16(da, sa, cnt)
                cnt = compact16(dbv, sbv, cnt)
                return cnt

            return lax.fori_loop(0, VPB, filt, cnt)

        pltpu.async_copy(src_hbm.at[pl.ds(ebase, BLK)], srcb0, sem_e0)
        pltpu.async_copy(dst_hbm.at[pl.ds(ebase, BLK)], dstb0, sem_e0)

        def pair_block(pb, cnt):
            base0 = ebase + 2 * pb * BLK
            base1 = base0 + BLK
            pltpu.make_async_copy(src_hbm.at[pl.ds(base0, BLK)], srcb0,
                                  sem_e0).wait()
            pltpu.make_async_copy(dst_hbm.at[pl.ds(base0, BLK)], dstb0,
                                  sem_e0).wait()
            pltpu.async_copy(src_hbm.at[pl.ds(base1, BLK)], srcb1, sem_e1)
            pltpu.async_copy(dst_hbm.at[pl.ds(base1, BLK)], dstb1, sem_e1)
            cnt = filter_block(srcb0, dstb0, cnt)
            npair = cnt // PAIR
            fire_pairs(npair)
            cnt = carry_remainder(cnt, npair)
            pltpu.make_async_copy(src_hbm.at[pl.ds(base1, BLK)], srcb1,
                                  sem_e1).wait()
            pltpu.make_async_copy(dst_hbm.at[pl.ds(base1, BLK)], dstb1,
                                  sem_e1).wait()

            @pl.when(pb < NBLK // 2 - 1)
            def _():
                base2 = base1 + BLK
                pltpu.async_copy(src_hbm.at[pl.ds(base2, BLK)], srcb0,
                                 sem_e0)
                pltpu.async_copy(dst_hbm.at[pl.ds(base2, BLK)], dstb0,
                                 sem_e0)

            cnt = filter_block(srcb1, dstb1, cnt)
            npair = cnt // PAIR
            fire_pairs(npair)
            cnt = carry_remainder(cnt, npair)
            return cnt

        cnt = lax.fori_loop(0, NBLK // 2, pair_block, jnp.int32(0))

        sent_d = jnp.full((LANES,), C, jnp.int32)
        sent_s = jnp.zeros((LANES,), jnp.int32)
        for j in range(PAIR // LANES):
            cdst[pl.ds(cnt + j * LANES, LANES)] = sent_d
            csrc[pl.ds(cnt + j * LANES, LANES)] = sent_s
        fire_pairs((cnt + PAIR) // PAIR)
        plsc.subcore_barrier()

        pltpu.sync_copy(acc.at[pl.ds(sub * RPT, RPT)],
                        out_hbm.at[pl.ds(lo + sub * RPT, RPT)])
        plsc.subcore_barrier()


def kernel(hidden_states, adj_i, adj_ii, adj_iii, adj_a,
           W_i, W_ii, W_iii, W_a):
    w_cat = jnp.concatenate([W_i, W_ii, W_iii, W_a], axis=1)
    h = _matmul(hidden_states, w_cat)
    h4 = h.reshape(N * 4, D)
    srcs = jnp.concatenate([adj_i[0], adj_ii[0], adj_iii[0], adj_a[0]])
    dsts = jnp.concatenate([adj_i[1], adj_ii[1], adj_iii[1], adj_a[1]])
    zeros = jnp.zeros((C, D), jnp.float32)
    out = _edge_aggregate(h4, srcs, dsts, zeros)
    return out[:N]
